# baseline (speedup 1.0000x reference)
import functools

import jax
import jax.numpy as jnp
from jax import lax
from jax.experimental import pallas as pl
from jax.experimental.pallas import tpu as pltpu
from jax.experimental.pallas import tpu_sc as plsc

B = 4
S = 4096
D = 2048
NC = 2
NS = 16
NW = NC * NS
S_PER_W = S // NW
C = 8
NCHUNK = S_PER_W // C
NITEMS = NCHUNK * B
NBUF = 5

_MESH = plsc.VectorSubcoreMesh(core_axis_name="c", subcore_axis_name="s")


@functools.partial(
    pl.kernel,
    out_type=jax.ShapeDtypeStruct((B * S, D), jnp.float32),
    mesh=_MESH,
    scratch_types=[
        pltpu.VMEM((B, S_PER_W), jnp.int32),
        [pltpu.VMEM((C, D), jnp.float32) for _ in range(2)],
        [pltpu.VMEM((C, D), jnp.float32) for _ in range(NBUF)],
        [pltpu.SemaphoreType.DMA for _ in range(NBUF)],
        [pltpu.SemaphoreType.DMA for _ in range(NBUF)],
        [pltpu.SemaphoreType.DMA for _ in range(2)],
    ],
)
def _embed(x_hbm, word_hbm, pos_hbm, out_hbm,
           idx_all, pos, acc, gsem, wsem, psem):
    wid = lax.axis_index("s") * NC + lax.axis_index("c")
    s0 = wid * S_PER_W

    def flat_base(k):
        ci, b = divmod(k, B)
        return b * S + s0 + ci * C

    def idx_ref(k):
        ci, b = divmod(k, B)
        return idx_all.at[b, pl.ds(ci * C, C)]

    def fire_gather(k, slot):
        pltpu.async_copy(word_hbm.at[idx_ref(k)], acc[slot], gsem[slot])

    def wait_gather(k, slot):
        pltpu.make_async_copy(word_hbm.at[idx_ref(k)], acc[slot],
                              gsem[slot]).wait()

    def fire_write(k, slot):
        pltpu.async_copy(acc[slot], out_hbm.at[pl.ds(flat_base(k), C)],
                         wsem[slot])

    def wait_write(k, slot):
        pltpu.make_async_copy(acc[slot], out_hbm.at[pl.ds(flat_base(k), C)],
                              wsem[slot]).wait()

    def fire_pos(ci):
        pltpu.async_copy(pos_hbm.at[pl.ds(s0 + ci * C, C)], pos[ci % 2],
                         psem[ci % 2])

    def wait_pos(ci):
        pltpu.make_async_copy(pos_hbm.at[pl.ds(s0 + ci * C, C)], pos[ci % 2],
                              psem[ci % 2]).wait()

    for b in range(B):
        pltpu.async_copy(x_hbm.at[pl.ds(b * S + s0, S_PER_W)], idx_all.at[b],
                         psem[0])
    for b in range(B):
        pltpu.make_async_copy(x_hbm.at[pl.ds(b * S + s0, S_PER_W)],
                              idx_all.at[b], psem[0]).wait()

    fire_pos(0)
    fire_gather(0, 0)
    fire_gather(1, 1)
    fire_gather(2, 2)

    for k in range(NITEMS):
        ci, b = divmod(k, B)
        slot = k % NBUF
        if b == 0:
            wait_pos(ci)
            if ci + 1 < NCHUNK:
                fire_pos(ci + 1)
        if k + 3 < NITEMS:
            if k + 3 >= NBUF:
                wait_write(k + 3 - NBUF, (k + 3) % NBUF)
            fire_gather(k + 3, (k + 3) % NBUF)
        wait_gather(k, slot)
        pv = pos[ci % 2]

        @plsc.parallel_loop(0, C * D, 16, unroll=8)
        def _flat(i):
            r = i // D
            c = i - r * D
            plsc.addupdate(acc[slot].at[r, pl.ds(c, 16)],
                           pv[r, pl.ds(c, 16)])

        fire_write(k, slot)

    for j in range(max(0, NITEMS - NBUF), NITEMS):
        wait_write(j, j % NBUF)


def kernel(x, word_table, pos_table):
    out = _embed(x.reshape(B * S), word_table, pos_table)
    return out.reshape(B, S, D)


# speedup vs baseline: 1.0074x; 1.0074x over previous
import functools

import jax
import jax.numpy as jnp
from jax import lax
from jax.experimental import pallas as pl
from jax.experimental.pallas import tpu as pltpu
from jax.experimental.pallas import tpu_sc as plsc

B = 4
S = 4096
D = 2048
NC = 2
NS = 16
NW = NC * NS
S_PER_W = S // NW
C = 8
NCHUNK = S_PER_W // C
NITEMS = NCHUNK * B
NBUF = 4

_MESH = plsc.VectorSubcoreMesh(core_axis_name="c", subcore_axis_name="s")


@functools.partial(
    pl.kernel,
    out_type=jax.ShapeDtypeStruct((B * S, D), jnp.float32),
    mesh=_MESH,
    scratch_types=[
        pltpu.VMEM((B, S_PER_W), jnp.int32),
        [pltpu.VMEM((C, D), jnp.float32) for _ in range(2)],
        [pltpu.VMEM((C, D), jnp.float32) for _ in range(NBUF)],
        [pltpu.SemaphoreType.DMA for _ in range(NBUF)],
        [pltpu.SemaphoreType.DMA for _ in range(NBUF)],
        [pltpu.SemaphoreType.DMA for _ in range(2)],
    ],
)
def _embed(x_hbm, word_hbm, pos_hbm, out_hbm,
           idx_all, pos, acc, gsem, wsem, psem):
    wid = lax.axis_index("s") * NC + lax.axis_index("c")
    s0 = wid * S_PER_W

    def flat_base(k):
        ci, b = divmod(k, B)
        return b * S + s0 + ci * C

    def idx_ref(k):
        ci, b = divmod(k, B)
        return idx_all.at[b, pl.ds(ci * C, C)]

    def fire_gather(k, slot):
        pltpu.async_copy(word_hbm.at[idx_ref(k)], acc[slot], gsem[slot])

    def wait_gather(k, slot):
        pltpu.make_async_copy(word_hbm.at[idx_ref(k)], acc[slot],
                              gsem[slot]).wait()

    def fire_write(k, slot):
        pltpu.async_copy(acc[slot], out_hbm.at[pl.ds(flat_base(k), C)],
                         wsem[slot])

    def wait_write(k, slot):
        pltpu.make_async_copy(acc[slot], out_hbm.at[pl.ds(flat_base(k), C)],
                              wsem[slot]).wait()

    def fire_pos(ci):
        pltpu.async_copy(pos_hbm.at[pl.ds(s0 + ci * C, C)], pos[ci % 2],
                         psem[ci % 2])

    def wait_pos(ci):
        pltpu.make_async_copy(pos_hbm.at[pl.ds(s0 + ci * C, C)], pos[ci % 2],
                              psem[ci % 2]).wait()

    for b in range(B):
        pltpu.async_copy(x_hbm.at[pl.ds(b * S + s0, S_PER_W)], idx_all.at[b],
                         psem[0])
    for b in range(B):
        pltpu.make_async_copy(x_hbm.at[pl.ds(b * S + s0, S_PER_W)],
                              idx_all.at[b], psem[0]).wait()

    fire_pos(0)
    fire_gather(0, 0)
    fire_gather(1, 1)

    for k in range(NITEMS):
        ci, b = divmod(k, B)
        slot = k % NBUF
        if b == 0:
            wait_pos(ci)
            if ci + 1 < NCHUNK:
                fire_pos(ci + 1)
        if k + 2 < NITEMS:
            if k + 2 >= NBUF:
                wait_write(k + 2 - NBUF, (k + 2) % NBUF)
            fire_gather(k + 2, (k + 2) % NBUF)
        wait_gather(k, slot)
        pv = pos[ci % 2]

        @plsc.parallel_loop(0, C * D, 16, unroll=8)
        def _flat(i):
            r = i // D
            c = i - r * D
            plsc.addupdate(acc[slot].at[r, pl.ds(c, 16)],
                           pv[r, pl.ds(c, 16)])

        fire_write(k, slot)

    for j in range(max(0, NITEMS - NBUF), NITEMS):
        wait_write(j, j % NBUF)


def kernel(x, word_table, pos_table):
    out = _embed(x.reshape(B * S), word_table, pos_table)
    return out.reshape(B, S, D)
